# baseline (speedup 1.0000x reference)
import functools

import jax
import jax.numpy as jnp
from jax.experimental import pallas as pl
from jax.experimental.pallas import tpu as pltpu

B, T, C = 1, 2048, 768
E = 8
K = 2
H = 1024
N = B * T
NK = N * K
BM = 256
NPAD = NK + E * BM
NBLK = NPAD // BM

EPAD = 128


def _router_body(x_ref, wr_ref, br_ref, logits_ref, idx_ref, sc_ref):
    x = x_ref[...]
    wr = wr_ref[...]
    lg = jnp.dot(x, wr, preferred_element_type=jnp.float32) + br_ref[...]
    logits_ref[...] = lg
    lane = jax.lax.broadcasted_iota(jnp.int32, (T, EPAD), 1)
    neg = jnp.float32(-1e30)
    lgm = jnp.where(lane < E, lg, neg)
    v0 = jnp.max(lgm, axis=1, keepdims=True)
    i0 = jnp.argmax(lgm, axis=1).reshape(T, 1)
    lgm2 = jnp.where(lane == i0, neg, lgm)
    v1 = jnp.max(lgm2, axis=1, keepdims=True)
    i1 = jnp.argmax(lgm2, axis=1).reshape(T, 1)
    e1 = jnp.exp(v1 - v0)
    s0 = 1.0 / (1.0 + e1)
    s1 = 1.0 - s0
    idx_ref[...] = jnp.where(lane == 0, i0, i1)
    sc_ref[...] = jnp.where(lane == 0, s0, s1)


def _router(x2d, wr_pad, br_pad):
    return pl.pallas_call(
        _router_body,
        out_shape=(
            jax.ShapeDtypeStruct((T, EPAD), jnp.float32),
            jax.ShapeDtypeStruct((T, EPAD), jnp.int32),
            jax.ShapeDtypeStruct((T, EPAD), jnp.float32),
        ),
    )(x2d, wr_pad, br_pad)


def _ffn_body(be_ref, act_ref, xs_ref, w1_ref, b1_ref, w2_ref, b2_ref,
              ss_ref, ys_ref):
    j = pl.program_id(0)

    @pl.when(act_ref[j] == 1)
    def _():
        xb = xs_ref[...]
        h = jnp.dot(xb, w1_ref[0], preferred_element_type=jnp.float32)
        h = jnp.maximum(h + b1_ref[0], 0.0).astype(jnp.bfloat16)
        y = jnp.dot(h, w2_ref[0], preferred_element_type=jnp.float32)
        ys_ref[...] = (y + b2_ref[0]) * ss_ref[...]


def _ffn(be, act, xs_pad, w1, b1, w2, b2, ss_pad):
    grid_spec = pltpu.PrefetchScalarGridSpec(
        num_scalar_prefetch=2,
        grid=(NBLK,),
        in_specs=[
            pl.BlockSpec((BM, C), lambda j, be, act: (j, 0)),
            pl.BlockSpec((1, C, H), lambda j, be, act: (be[j], 0, 0)),
            pl.BlockSpec((1, 1, H), lambda j, be, act: (be[j], 0, 0)),
            pl.BlockSpec((1, H, C), lambda j, be, act: (be[j], 0, 0)),
            pl.BlockSpec((1, 1, C), lambda j, be, act: (be[j], 0, 0)),
            pl.BlockSpec((BM, 1), lambda j, be, act: (j, 0)),
        ],
        out_specs=pl.BlockSpec((BM, C), lambda j, be, act: (j, 0)),
    )
    return pl.pallas_call(
        _ffn_body,
        grid_spec=grid_spec,
        out_shape=jax.ShapeDtypeStruct((NPAD, C), jnp.float32),
    )(be, act, xs_pad, w1, b1, w2, b2, ss_pad)


@jax.jit
def kernel(x, Wr, br, W1, b1, W2, b2):
    x2d = x.reshape(N, C)
    wr_pad = jnp.zeros((C, EPAD), jnp.float32).at[:, :E].set(Wr)
    br_pad = jnp.zeros((1, EPAD), jnp.float32).at[0, :E].set(br)

    logits_p, idx_p, sc_p = _router(x2d, wr_pad, br_pad)
    logits = logits_p[:, :E].reshape(B, T, E)
    topk_idx = idx_p[:, :K].reshape(B, T, K)
    scores = sc_p[:, :K]

    e_flat = idx_p[:, :K].reshape(NK)
    onehot = (e_flat[:, None] == jnp.arange(E, dtype=jnp.int32)[None, :])
    counts = jnp.sum(onehot.astype(jnp.int32), axis=0)
    pc = ((counts + BM - 1) // BM) * BM
    pstart = jnp.concatenate([jnp.zeros((1,), jnp.int32),
                              jnp.cumsum(pc)])[:E]
    rank = jnp.sum(onehot.astype(jnp.int32) *
                   jnp.cumsum(onehot.astype(jnp.int32), axis=0), axis=1) - 1
    pos = pstart[e_flat] + rank
    rows_pad = jnp.zeros((NPAD,), jnp.int32).at[pos].set(
        jnp.arange(NK, dtype=jnp.int32) // K)
    ss_pad = jnp.zeros((NPAD, 1), jnp.float32).at[pos, 0].set(
        scores.reshape(NK))
    inv = pos.reshape(N, K)
    tot = jnp.sum(pc)
    blk_starts = jnp.arange(NBLK, dtype=jnp.int32) * BM
    be = jnp.searchsorted(jnp.cumsum(pc), blk_starts, side='right')
    be = jnp.clip(be, 0, E - 1).astype(jnp.int32)
    act = (blk_starts < tot).astype(jnp.int32)

    xs_pad = x2d.astype(jnp.bfloat16)[rows_pad]
    w1b = W1.astype(jnp.bfloat16)
    w2b = W2.astype(jnp.bfloat16)
    ys = _ffn(be, act, xs_pad, w1b, b1.reshape(E, 1, H),
              w2b, b2.reshape(E, 1, C), ss_pad)
    out2d = ys[inv[:, 0]] + ys[inv[:, 1]]
    return (logits, topk_idx, out2d.reshape(B, T, C))


# speedup vs baseline: 1.9914x; 1.9914x over previous
import functools

import jax
import jax.numpy as jnp
from jax import lax
from jax.experimental import pallas as pl
from jax.experimental.pallas import tpu as pltpu
from jax.experimental.pallas import tpu_sc as plsc

B, T, C = 1, 2048, 768
E = 8
K = 2
H = 1024
N = B * T
NK = N * K
BM = 256
NPAD = NK + E * BM
NBLK = NPAD // BM

NW = 32
TPW = N // NW
SUB = 32
LANES = 16


def _router_body(x_ref, wr_ref, br_ref, logits_ref, idx_ref, aux_ref,
                 meta_ref):
    x = x_ref[...]
    lg = jnp.dot(x, wr_ref[...], preferred_element_type=jnp.float32)
    lg = lg + br_ref[...]
    logits_ref[...] = lg
    lane = lax.broadcasted_iota(jnp.int32, (T, E), 1)
    neg = jnp.float32(-1e30)
    v0 = jnp.max(lg, axis=1, keepdims=True)
    i0 = jnp.argmax(lg, axis=1).reshape(T, 1)
    lg2 = jnp.where(lane == i0, neg, lg)
    v1 = jnp.max(lg2, axis=1, keepdims=True)
    i1 = jnp.argmax(lg2, axis=1).reshape(T, 1)
    e1 = jnp.exp(v1 - v0)
    s0 = 1.0 / (1.0 + e1)
    s1 = 1.0 - s0
    idx_ref[...] = jnp.where(lane == 0, i0, i1)

    c0 = (lane == i0).astype(jnp.float32)
    c1 = (lane == i1).astype(jnp.float32)
    s = c0 + c1
    inc = s
    k = 1
    while k < T:
        shifted = jnp.concatenate(
            [jnp.zeros((k, E), jnp.float32), inc[: T - k, :]], axis=0)
        inc = inc + shifted
        k *= 2
    ex = inc - s
    counts = jnp.sum(s, axis=0, keepdims=True)
    pc = jnp.floor((counts + (BM - 1)) / BM) * BM
    triu = (lax.broadcasted_iota(jnp.int32, (E, E), 0) <
            lax.broadcasted_iota(jnp.int32, (E, E), 1)).astype(jnp.float32)
    pstart = jnp.dot(pc, triu, preferred_element_type=jnp.float32)
    cum_pc = pstart + pc
    tot = jnp.sum(pc, axis=1, keepdims=True)

    rank0 = jnp.sum(c0 * ex, axis=1, keepdims=True)
    rank1 = jnp.sum(c1 * (ex + c0), axis=1, keepdims=True)
    base0 = jnp.sum(c0 * pstart, axis=1, keepdims=True)
    base1 = jnp.sum(c1 * pstart, axis=1, keepdims=True)
    pos0 = base0 + rank0
    pos1 = base1 + rank1
    aux = jnp.where(lane == 0, pos0,
                    jnp.where(lane == 1, pos1,
                              jnp.where(lane == 2, s0, s1)))
    aux_ref[...] = aux

    jgrid = lax.broadcasted_iota(jnp.int32, (128, E), 0) * BM
    cum_i = jnp.broadcast_to(cum_pc.astype(jnp.int32), (128, E))
    be = jnp.sum((cum_i <= jgrid).astype(jnp.int32), axis=1, keepdims=True)
    be = jnp.minimum(be, E - 1)
    act = (jgrid[:, :1] < tot.astype(jnp.int32)).astype(jnp.int32)
    mlane = lax.broadcasted_iota(jnp.int32, (128, E), 1)
    meta_ref[...] = jnp.where(mlane == 0, be, act)


def _router(x2d, wr, br2d):
    return pl.pallas_call(
        _router_body,
        out_shape=(
            jax.ShapeDtypeStruct((T, E), jnp.float32),
            jax.ShapeDtypeStruct((T, E), jnp.int32),
            jax.ShapeDtypeStruct((T, E), jnp.float32),
            jax.ShapeDtypeStruct((128, E), jnp.int32),
        ),
    )(x2d, wr, br2d)


def _dispatch_body(x_hbm, post_hbm, xs_hbm, idx0_v, idx1_v, rows_v, sem):
    wid = lax.axis_index("s") * 2 + lax.axis_index("c")
    base = wid * TPW
    cp = pltpu.async_copy(x_hbm.at[pl.ds(base, TPW)], rows_v, sem)
    pltpu.sync_copy(post_hbm.at[0, pl.ds(base, TPW)], idx0_v)
    pltpu.sync_copy(post_hbm.at[1, pl.ds(base, TPW)], idx1_v)
    cp.wait()
    s0 = pltpu.async_copy(rows_v, xs_hbm.at[idx0_v], sem)
    s1 = pltpu.async_copy(rows_v, xs_hbm.at[idx1_v], sem)
    s0.wait()
    s1.wait()


def _dispatch(x2d, post):
    mesh = plsc.VectorSubcoreMesh(core_axis_name="c", subcore_axis_name="s")
    return pl.kernel(
        _dispatch_body,
        out_type=jax.ShapeDtypeStruct((NPAD, C), jnp.float32),
        mesh=mesh,
        scratch_types=[
            pltpu.VMEM((TPW,), jnp.int32),
            pltpu.VMEM((TPW,), jnp.int32),
            pltpu.VMEM((TPW, C), jnp.float32),
            pltpu.SemaphoreType.DMA,
        ],
    )(x2d, post)


def _ffn_body(be_ref, act_ref, xs_ref, w1_ref, b1_ref, w2_ref, b2_ref,
              ys_ref):
    j = pl.program_id(0)

    @pl.when(act_ref[j] == 1)
    def _():
        xb = xs_ref[...].astype(jnp.bfloat16)
        w1 = w1_ref[0].astype(jnp.bfloat16)
        h = jnp.dot(xb, w1, preferred_element_type=jnp.float32)
        h = jnp.maximum(h + b1_ref[0], 0.0).astype(jnp.bfloat16)
        w2 = w2_ref[0].astype(jnp.bfloat16)
        y = jnp.dot(h, w2, preferred_element_type=jnp.float32)
        ys_ref[...] = y + b2_ref[0]


def _ffn(be, act, xs_pad, w1, b1, w2, b2):
    grid_spec = pltpu.PrefetchScalarGridSpec(
        num_scalar_prefetch=2,
        grid=(NBLK,),
        in_specs=[
            pl.BlockSpec((BM, C), lambda j, be, act: (j, 0)),
            pl.BlockSpec((1, C, H), lambda j, be, act: (be[j], 0, 0)),
            pl.BlockSpec((1, 1, H), lambda j, be, act: (be[j], 0, 0)),
            pl.BlockSpec((1, H, C), lambda j, be, act: (be[j], 0, 0)),
            pl.BlockSpec((1, 1, C), lambda j, be, act: (be[j], 0, 0)),
        ],
        out_specs=pl.BlockSpec((BM, C), lambda j, be, act: (j, 0)),
    )
    return pl.pallas_call(
        _ffn_body,
        grid_spec=grid_spec,
        out_shape=jax.ShapeDtypeStruct((NPAD, C), jnp.float32),
    )(be, act, xs_pad, w1, b1, w2, b2)


def _combine_body(ys_hbm, post_hbm, sbc_hbm, out_hbm, sb_v, idx0_v, idx1_v,
                  r0_v, r1_v, o_v, sem):
    wid = lax.axis_index("s") * 2 + lax.axis_index("c")
    for p in range(TPW // SUB):
        base = wid * SUB + p * (NW * SUB)
        pltpu.sync_copy(post_hbm.at[0, pl.ds(base, SUB)], idx0_v)
        pltpu.sync_copy(post_hbm.at[1, pl.ds(base, SUB)], idx1_v)
        pltpu.sync_copy(sbc_hbm.at[pl.ds(base, SUB)], sb_v)
        cp0 = pltpu.async_copy(ys_hbm.at[idx0_v], r0_v, sem)
        cp1 = pltpu.async_copy(ys_hbm.at[idx1_v], r1_v, sem)
        cp0.wait()
        cp1.wait()

        def tok(r, carry):
            s0 = sb_v[r, pl.ds(0, LANES)]
            s1 = sb_v[r, pl.ds(LANES, LANES)]
            for cc in range(C // LANES):
                a = r0_v[r, pl.ds(cc * LANES, LANES)]
                b = r1_v[r, pl.ds(cc * LANES, LANES)]
                o_v[r, pl.ds(cc * LANES, LANES)] = a * s0 + b * s1
            return carry

        lax.fori_loop(0, SUB, tok, 0)
        pltpu.sync_copy(o_v, out_hbm.at[pl.ds(base, SUB)])


def _combine(ys, post, sbc):
    mesh = plsc.VectorSubcoreMesh(core_axis_name="c", subcore_axis_name="s")
    return pl.kernel(
        _combine_body,
        out_type=jax.ShapeDtypeStruct((N, C), jnp.float32),
        mesh=mesh,
        scratch_types=[
            pltpu.VMEM((SUB, 2 * LANES), jnp.float32),
            pltpu.VMEM((SUB,), jnp.int32),
            pltpu.VMEM((SUB,), jnp.int32),
            pltpu.VMEM((SUB, C), jnp.float32),
            pltpu.VMEM((SUB, C), jnp.float32),
            pltpu.VMEM((SUB, C), jnp.float32),
            pltpu.SemaphoreType.DMA,
        ],
    )(ys, post, sbc)


@jax.jit
def kernel(x, Wr, br, W1, b1, W2, b2):
    x2d = x.reshape(N, C)
    logits, idx8, aux, meta = _router(x2d, Wr, br.reshape(1, E))
    topk_idx = idx8[:, :K].reshape(B, T, K)
    be = meta[:NBLK, 0]
    act = meta[:NBLK, 1]
    post = aux[:, :K].astype(jnp.int32).T
    sbc = jnp.concatenate(
        [jnp.broadcast_to(aux[:, 2:3], (N, LANES)),
         jnp.broadcast_to(aux[:, 3:4], (N, LANES))], axis=1)
    xs_pad = _dispatch(x2d, post)
    ys = _ffn(be, act, xs_pad, W1, b1.reshape(E, 1, H), W2,
              b2.reshape(E, 1, C))
    out2d = _combine(ys, post, sbc)
    return (logits.reshape(B, T, E), topk_idx, out2d.reshape(B, T, C))
